# baseline (speedup 1.0000x reference)
import functools

import jax
import jax.numpy as jnp
from jax import lax
from jax.experimental import pallas as pl
from jax.experimental.pallas import tpu as pltpu
from jax.experimental.pallas import tpu_sc as plsc

NC = 2
NS = 16
NW = NC * NS
L = 16
NSLOT = 3

BIG = 3.0e38


def _make_sc_call(E, C, N, B):
    n_chunks = E // B
    n_rounds = (n_chunks + NW - 1) // NW
    n_loop = -(-(n_rounds + 2) // NSLOT) * NSLOT
    tiles_per_chunk = B // 128
    rc_rows = 2 * tiles_per_chunk

    mesh = plsc.VectorSubcoreMesh(core_axis_name="c", subcore_axis_name="s")

    @functools.partial(
        pl.kernel,
        mesh=mesh,
        compiler_params=pltpu.CompilerParams(
            needs_layout_passes=False,
        ),
        out_type=(
            jax.ShapeDtypeStruct((C, E), jnp.float32),
            jax.ShapeDtypeStruct((NW * L,), jnp.float32),
        ),
        scratch_types=[
            pltpu.VMEM((N,), jnp.int32),
            *[pltpu.VMEM((rc_rows, 128), jnp.int32) for _ in range(NSLOT)],
            *[pltpu.VMEM((C, B), jnp.float32) for _ in range(NSLOT)],
            pltpu.VMEM((L,), jnp.float32),
            *[pltpu.SemaphoreType.DMA for _ in range(2 * NSLOT)],
        ],
    )
    def sc_call(bl_hbm, ei_hbm, atom_hbm, out_hbm, viol_hbm,
                atom_v, rc0, rc1, rc2, bl0, bl1, bl2, acc_v,
                isem0, isem1, isem2, osem0, osem1, osem2):
        rc_v = (rc0, rc1, rc2)
        bl_v = (bl0, bl1, bl2)
        isem = (isem0, isem1, isem2)
        osem = (osem0, osem1, osem2)

        cid = lax.axis_index("c")
        sid = lax.axis_index("s")
        wid = sid * NC + cid

        pltpu.sync_copy(atom_hbm, atom_v)

        def in_copies(chunk, s):
            return (
                pltpu.make_async_copy(
                    ei_hbm.at[pl.ds(chunk * rc_rows, rc_rows), :],
                    rc_v[s], isem[s]),
                pltpu.make_async_copy(
                    bl_hbm.at[pl.ds(1, 4), pl.ds(chunk * B, B)],
                    bl_v[s].at[pl.ds(1, 4), :], isem[s]),
            )

        def out_copy(chunk, s):
            return pltpu.make_async_copy(
                bl_v[s].at[pl.ds(1, 4), :],
                out_hbm.at[pl.ds(1, 4), pl.ds(chunk * B, B)], osem[s])

        def col0_copy(chunk, s):
            return pltpu.make_async_copy(
                bl_hbm.at[pl.ds(0, 1), pl.ds(chunk * B, B)],
                out_hbm.at[pl.ds(0, 1), pl.ds(chunk * B, B)], osem[s])

        def issue_in(ri, s):
            chunk = ri * NW + wid

            @pl.when((ri < n_rounds) & (chunk < n_chunks))
            def _():
                for c in in_copies(chunk, s):
                    c.start()

        def wait_in(ri, s):
            chunk = ri * NW + wid

            @pl.when((ri < n_rounds) & (chunk < n_chunks))
            def _():
                for c in in_copies(chunk, s):
                    c.wait()

        def issue_out(ri, s):
            chunk = ri * NW + wid

            @pl.when((ri < n_rounds) & (chunk < n_chunks))
            def _():
                out_copy(chunk, s).start()
                col0_copy(chunk, s).start()

        def wait_out(ri, s):
            chunk = ri * NW + wid

            @pl.when((ri >= 0) & (ri < n_rounds) & (chunk < n_chunks))
            def _():
                out_copy(chunk, s).wait()
                col0_copy(chunk, s).wait()

        def compute(ri, s, acc):
            chunk = ri * NW + wid

            def grp_outer(tt, acc):
                for gg in range(8):
                    off = gg * L
                    r = rc_v[s][2 * tt, pl.ds(off, L)]
                    c = rc_v[s][2 * tt + 1, pl.ds(off, L)]
                    t1 = plsc.load_gather(atom_v, [r])
                    t2 = plsc.load_gather(atom_v, [c])
                    is4 = (t1 == 4) | (t2 == 4)
                    is5 = (t1 == 5) | (t2 == 5)
                    sub4 = jnp.where(is4, 100.0, 0.0).astype(jnp.float32)
                    sub45 = sub4 + jnp.where(is5, 50.0, 0.0).astype(
                        jnp.float32)
                    s4x = jnp.where(is4, 100.0, BIG).astype(jnp.float32)
                    zero = jnp.zeros((L,), jnp.float32)
                    boff = tt * 128 + off
                    gacc = None
                    for ci in range(1, C):
                        x = bl_v[s][ci, pl.ds(boff, L)]
                        v = jnp.maximum(x - s4x, zero)
                        gacc = v if gacc is None else gacc + v
                        bl_v[s][ci, pl.ds(boff, L)] = (
                            x - sub4 if ci == 1 else x - sub45)
                    acc = acc + gacc
                return acc

            return lax.cond(
                (ri < n_rounds) & (chunk < n_chunks),
                lambda a: lax.fori_loop(0, tiles_per_chunk, grp_outer, a),
                lambda a: a,
                acc,
            )

        issue_in(jnp.int32(0), 0)

        def loop_body(rp, acc):
            for b in range(NSLOT):
                ri = rp * NSLOT + b
                s_next = (b + 1) % NSLOT
                wait_out(ri - 2, s_next)
                issue_in(ri + 1, s_next)
                wait_in(ri, b)
                acc = compute(ri, b, acc)
                issue_out(ri, b)
            return acc

        acc = lax.fori_loop(0, n_loop // NSLOT, loop_body,
                            jnp.zeros((L,), jnp.float32))
        acc_v[...] = acc
        pltpu.sync_copy(acc_v, viol_hbm.at[pl.ds(wid * L, L)])

    return sc_call


def kernel(bond_logits, edge_index, atom_types):
    E, C = bond_logits.shape
    N = atom_types.shape[0]
    B = 1024
    assert E % B == 0 and B % 512 == 0
    ei_view = edge_index.reshape(2, E // 128, 128).swapaxes(0, 1).reshape(
        2 * E // 128, 128)
    sc_call = _make_sc_call(E, C, N, B)
    out_t, partials = sc_call(bond_logits.T, ei_view, atom_types)
    return out_t.T, jnp.sum(partials) / E


# speedup vs baseline: 3.0592x; 3.0592x over previous
import functools

import jax
import jax.numpy as jnp
from jax import lax
from jax.experimental import pallas as pl
from jax.experimental.pallas import tpu as pltpu
from jax.experimental.pallas import tpu_sc as plsc

NC = 2
NS = 16
NW = NC * NS
L = 16
NSLOT = 3

BIG = 3.0e38


def _make_sc_call(E, C, N, B):
    n_chunks = E // B
    n_rounds = (n_chunks + NW - 1) // NW
    n_loop = -(-(n_rounds + 2) // NSLOT) * NSLOT
    tiles_per_chunk = B // 128
    rc_rows = 2 * tiles_per_chunk

    mesh = plsc.VectorSubcoreMesh(core_axis_name="c", subcore_axis_name="s")

    @functools.partial(
        pl.kernel,
        mesh=mesh,
        compiler_params=pltpu.CompilerParams(
            needs_layout_passes=False,
        ),
        out_type=(
            jax.ShapeDtypeStruct((C, E), jnp.float32),
            jax.ShapeDtypeStruct((NW * L,), jnp.float32),
        ),
        scratch_types=[
            pltpu.VMEM((N,), jnp.int32),
            *[pltpu.VMEM((rc_rows, 128), jnp.int32) for _ in range(NSLOT)],
            *[pltpu.VMEM((C, B), jnp.float32) for _ in range(NSLOT)],
            pltpu.VMEM((L,), jnp.float32),
            *[pltpu.SemaphoreType.DMA for _ in range(2 * NSLOT)],
        ],
    )
    def sc_call(bl_hbm, ei_hbm, atom_hbm, out_hbm, viol_hbm,
                atom_v, rc0, rc1, rc2, bl0, bl1, bl2, acc_v,
                isem0, isem1, isem2, osem0, osem1, osem2):
        rc_v = (rc0, rc1, rc2)
        bl_v = (bl0, bl1, bl2)
        isem = (isem0, isem1, isem2)
        osem = (osem0, osem1, osem2)

        cid = lax.axis_index("c")
        sid = lax.axis_index("s")
        wid = sid * NC + cid

        pltpu.sync_copy(atom_hbm, atom_v)

        def in_copies(chunk, s):
            return (
                pltpu.make_async_copy(
                    ei_hbm.at[pl.ds(chunk * rc_rows, rc_rows), :],
                    rc_v[s], isem[s]),
                pltpu.make_async_copy(
                    bl_hbm.at[:, pl.ds(chunk * B, B)], bl_v[s], isem[s]),
            )

        def out_copy(chunk, s):
            return pltpu.make_async_copy(
                bl_v[s], out_hbm.at[:, pl.ds(chunk * B, B)], osem[s])

        def issue_in(ri, s):
            chunk = ri * NW + wid

            @pl.when((ri < n_rounds) & (chunk < n_chunks))
            def _():
                for c in in_copies(chunk, s):
                    c.start()

        def wait_in(ri, s):
            chunk = ri * NW + wid

            @pl.when((ri < n_rounds) & (chunk < n_chunks))
            def _():
                for c in in_copies(chunk, s):
                    c.wait()

        def issue_out(ri, s):
            chunk = ri * NW + wid

            @pl.when((ri < n_rounds) & (chunk < n_chunks))
            def _():
                out_copy(chunk, s).start()

        def wait_out(ri, s):
            chunk = ri * NW + wid

            @pl.when((ri >= 0) & (ri < n_rounds) & (chunk < n_chunks))
            def _():
                out_copy(chunk, s).wait()

        def compute(ri, s, acc):
            chunk = ri * NW + wid

            def grp_outer(tt, acc):
                for gg in range(8):
                    off = gg * L
                    r = rc_v[s][2 * tt, pl.ds(off, L)]
                    c = rc_v[s][2 * tt + 1, pl.ds(off, L)]
                    t1 = plsc.load_gather(atom_v, [r])
                    t2 = plsc.load_gather(atom_v, [c])
                    is4 = (t1 == 4) | (t2 == 4)
                    is5 = (t1 == 5) | (t2 == 5)
                    sub4 = jnp.where(is4, 100.0, 0.0).astype(jnp.float32)
                    sub45 = sub4 + jnp.where(is5, 50.0, 0.0).astype(
                        jnp.float32)
                    s4x = jnp.where(is4, 100.0, BIG).astype(jnp.float32)
                    zero = jnp.zeros((L,), jnp.float32)
                    boff = tt * 128 + off
                    gacc = None
                    for ci in range(1, C):
                        x = bl_v[s][ci, pl.ds(boff, L)]
                        v = jnp.maximum(x - s4x, zero)
                        gacc = v if gacc is None else gacc + v
                        bl_v[s][ci, pl.ds(boff, L)] = (
                            x - sub4 if ci == 1 else x - sub45)
                    acc = acc + gacc
                return acc

            return lax.cond(
                (ri < n_rounds) & (chunk < n_chunks),
                lambda a: lax.fori_loop(0, tiles_per_chunk, grp_outer, a),
                lambda a: a,
                acc,
            )

        issue_in(jnp.int32(0), 0)

        def loop_body(rp, acc):
            for b in range(NSLOT):
                ri = rp * NSLOT + b
                s_next = (b + 1) % NSLOT
                wait_out(ri - 2, s_next)
                issue_in(ri + 1, s_next)
                wait_in(ri, b)
                acc = compute(ri, b, acc)
                issue_out(ri, b)
            return acc

        acc = lax.fori_loop(0, n_loop // NSLOT, loop_body,
                            jnp.zeros((L,), jnp.float32))
        acc_v[...] = acc
        pltpu.sync_copy(acc_v, viol_hbm.at[pl.ds(wid * L, L)])

    return sc_call


def kernel(bond_logits, edge_index, atom_types):
    E, C = bond_logits.shape
    N = atom_types.shape[0]
    B = 1024
    assert E % B == 0 and B % 512 == 0
    ei_view = edge_index.reshape(2, E // 128, 128).swapaxes(0, 1).reshape(
        2 * E // 128, 128)
    sc_call = _make_sc_call(E, C, N, B)
    out_t, partials = sc_call(bond_logits.T, ei_view, atom_types)
    return out_t.T, jnp.sum(partials) / E


# speedup vs baseline: 3.1335x; 1.0243x over previous
import functools

import jax
import jax.numpy as jnp
from jax import lax
from jax.experimental import pallas as pl
from jax.experimental.pallas import tpu as pltpu
from jax.experimental.pallas import tpu_sc as plsc

NC = 2
NS = 16
NW = NC * NS
L = 16
NSLOT = 3

BIG = 3.0e38


def _make_sc_call(E, C, N, B):
    n_chunks = E // B
    n_rounds = (n_chunks + NW - 1) // NW
    n_loop = -(-(n_rounds + NSLOT - 1) // NSLOT) * NSLOT
    tiles_per_chunk = B // 128
    rc_rows = 2 * tiles_per_chunk

    mesh = plsc.VectorSubcoreMesh(core_axis_name="c", subcore_axis_name="s")

    @functools.partial(
        pl.kernel,
        mesh=mesh,
        compiler_params=pltpu.CompilerParams(
            needs_layout_passes=False,
        ),
        out_type=(
            jax.ShapeDtypeStruct((C, E), jnp.float32),
            jax.ShapeDtypeStruct((NW * L,), jnp.float32),
        ),
        scratch_types=[
            pltpu.VMEM((N // 8,), jnp.int32),
            *[pltpu.VMEM((rc_rows, 128), jnp.int32) for _ in range(NSLOT)],
            *[pltpu.VMEM((C, B), jnp.float32) for _ in range(NSLOT)],
            pltpu.VMEM((L,), jnp.float32),
            *[pltpu.SemaphoreType.DMA for _ in range(2 * NSLOT)],
        ],
    )
    def sc_call(bl_hbm, ei_hbm, atom_hbm, out_hbm, viol_hbm,
                atom_v, rc0, rc1, rc2, bl0, bl1, bl2, acc_v,
                isem0, isem1, isem2, osem0, osem1, osem2):
        rc_v = (rc0, rc1, rc2)
        bl_v = (bl0, bl1, bl2)
        isem = (isem0, isem1, isem2)
        osem = (osem0, osem1, osem2)

        cid = lax.axis_index("c")
        sid = lax.axis_index("s")
        wid = sid * NC + cid

        pltpu.sync_copy(atom_hbm, atom_v)

        def in_copies(chunk, s):
            return (
                pltpu.make_async_copy(
                    ei_hbm.at[pl.ds(chunk * rc_rows, rc_rows), :],
                    rc_v[s], isem[s]),
                pltpu.make_async_copy(
                    bl_hbm.at[:, pl.ds(chunk * B, B)], bl_v[s], isem[s]),
            )

        def out_copy(chunk, s):
            return pltpu.make_async_copy(
                bl_v[s], out_hbm.at[:, pl.ds(chunk * B, B)], osem[s])

        def issue_in(ri, s):
            chunk = ri * NW + wid

            @pl.when((ri < n_rounds) & (chunk < n_chunks))
            def _():
                for c in in_copies(chunk, s):
                    c.start()

        def wait_in(ri, s):
            chunk = ri * NW + wid

            @pl.when((ri < n_rounds) & (chunk < n_chunks))
            def _():
                for c in in_copies(chunk, s):
                    c.wait()

        def issue_out(ri, s):
            chunk = ri * NW + wid

            @pl.when((ri < n_rounds) & (chunk < n_chunks))
            def _():
                out_copy(chunk, s).start()

        def wait_out(ri, s):
            chunk = ri * NW + wid

            @pl.when((ri >= 0) & (ri < n_rounds) & (chunk < n_chunks))
            def _():
                out_copy(chunk, s).wait()

        def compute(ri, s, acc):
            chunk = ri * NW + wid

            def grp_outer(tt, acc):
                for gg in range(8):
                    off = gg * L
                    r = rc_v[s][2 * tt, pl.ds(off, L)]
                    c = rc_v[s][2 * tt + 1, pl.ds(off, L)]
                    w1 = plsc.load_gather(
                        atom_v, [lax.shift_right_logical(r, 3)])
                    w2 = plsc.load_gather(
                        atom_v, [lax.shift_right_logical(c, 3)])
                    t1 = lax.shift_right_logical(
                        w1, lax.shift_left(r & 7, 2)) & 7
                    t2 = lax.shift_right_logical(
                        w2, lax.shift_left(c & 7, 2)) & 7
                    is4 = (t1 == 4) | (t2 == 4)
                    is5 = (t1 == 5) | (t2 == 5)
                    sub4 = jnp.where(is4, 100.0, 0.0).astype(jnp.float32)
                    sub45 = sub4 + jnp.where(is5, 50.0, 0.0).astype(
                        jnp.float32)
                    s4x = jnp.where(is4, 100.0, BIG).astype(jnp.float32)
                    zero = jnp.zeros((L,), jnp.float32)
                    boff = tt * 128 + off
                    gacc = None
                    for ci in range(1, C):
                        x = bl_v[s][ci, pl.ds(boff, L)]
                        v = jnp.maximum(x - s4x, zero)
                        gacc = v if gacc is None else gacc + v
                        bl_v[s][ci, pl.ds(boff, L)] = (
                            x - sub4 if ci == 1 else x - sub45)
                    acc = acc + gacc
                return acc

            return lax.cond(
                (ri < n_rounds) & (chunk < n_chunks),
                lambda a: lax.fori_loop(0, tiles_per_chunk, grp_outer, a),
                lambda a: a,
                acc,
            )

        issue_in(jnp.int32(0), 0)

        def loop_body(rp, acc):
            for b in range(NSLOT):
                ri = rp * NSLOT + b
                s_next = (b + 1) % NSLOT
                wait_out(ri + 1 - NSLOT, s_next)
                issue_in(ri + 1, s_next)
                wait_in(ri, b)
                acc = compute(ri, b, acc)
                issue_out(ri, b)
            return acc

        acc = lax.fori_loop(0, n_loop // NSLOT, loop_body,
                            jnp.zeros((L,), jnp.float32))
        acc_v[...] = acc
        pltpu.sync_copy(acc_v, viol_hbm.at[pl.ds(wid * L, L)])

    return sc_call


def kernel(bond_logits, edge_index, atom_types):
    E, C = bond_logits.shape
    N = atom_types.shape[0]
    B = 2560
    assert E % B == 0 and B % 512 == 0 and N % 8 == 0
    ei_view = edge_index.reshape(2, E // 128, 128).swapaxes(0, 1).reshape(
        2 * E // 128, 128)
    shifts = (4 * jnp.arange(8, dtype=jnp.int32))[None, :]
    atom_packed = jnp.sum(
        atom_types.reshape(N // 8, 8) << shifts, axis=1, dtype=jnp.int32)
    sc_call = _make_sc_call(E, C, N, B)
    out_t, partials = sc_call(bond_logits.T, ei_view, atom_packed)
    return out_t.T, jnp.sum(partials) / E
